# baseline (speedup 1.0000x reference)
import functools

import jax
import jax.numpy as jnp
from jax import lax
from jax.experimental import pallas as pl
from jax.experimental.pallas import tpu as pltpu
from jax.experimental.pallas import tpu_sc as plsc

EPS = 1e-5
CH = 128
NC = 2
NS = 16
D = 128


def _drain(acc, stage, out_ref, s, n_nodes):
  bpt = (n_nodes // NS) // 8 * 8
  rem = n_nodes - NS * bpt
  for k in range(0, bpt, CH):
    sz = min(CH, bpt - k)
    dyn = s * bpt + k
    pltpu.sync_copy(acc.at[pl.ds(dyn, sz)], stage.at[pl.ds(0, sz)])
    pltpu.sync_copy(stage.at[pl.ds(0, sz)], out_ref.at[pl.ds(dyn, sz)])
  if rem:
    @pl.when(s == NS - 1)
    def _():
      off = NS * bpt
      pltpu.sync_copy(acc.at[pl.ds(off, rem)], stage.at[pl.ds(0, rem)])
      pltpu.sync_copy(stage.at[pl.ds(0, rem)], out_ref.at[pl.ds(off, rem)])


def _hist_kernel(n_nodes, n_chunks):
  acc_rows = ((n_nodes + 16 + NS * 8 - 1) // (NS * 8)) * (NS * 8)
  rpt = acc_rows // NS
  cpt = n_chunks // (NC * NS)
  mesh = plsc.VectorSubcoreMesh(core_axis_name="c", subcore_axis_name="s")

  depth = 4
  @functools.partial(
      pl.kernel,
      out_type=jax.ShapeDtypeStruct((NC, n_nodes, D), jnp.float32),
      mesh=mesh,
      scratch_types=[
          pltpu.VMEM((cpt, 1, CH), jnp.int32),
          pltpu.VMEM((CH, D), jnp.float32),
          pltpu.VMEM_SHARED((acc_rows, D), jnp.float32),
          pltpu.SemaphoreType.DMA,
          pltpu.SemaphoreType.DMA,
      ],
  )
  def hist(zeros_hbm, ones_hbm, dst_hbm, out_hbm, idx_d, rows, acc,
           ssem, hsem):
    c = lax.axis_index("c")
    s = lax.axis_index("s")
    wid = s * NC + c
    ic = pltpu.async_copy(dst_hbm.at[pl.ds(wid * cpt, cpt)], idx_d, hsem)
    pltpu.sync_copy(zeros_hbm, rows)
    for k in range(0, rpt, CH):
      sz = min(CH, rpt - k)
      pltpu.sync_copy(rows.at[pl.ds(0, sz)],
                      acc.at[pl.ds(s * rpt + k, sz)])
    pltpu.sync_copy(ones_hbm, rows)
    ic.wait()
    plsc.subcore_barrier()

    def body(j, _):
      pltpu.async_copy(rows, acc.at[idx_d.at[j, 0]], ssem, add=True)
      @pl.when(j >= depth)
      def _():
        pltpu.make_async_copy(zeros_hbm, rows, ssem).wait()
      return 0
    lax.fori_loop(0, cpt, body, 0)
    for _ in range(depth):
      pltpu.make_async_copy(zeros_hbm, rows, ssem).wait()
    plsc.subcore_barrier()
    _drain(acc, rows, out_hbm.at[c], s, n_nodes)

  return hist


NBUF = 4


def _agg_kernel(n_nodes, n_chunks):
  acc_rows = ((n_nodes + 16 + NS * 8 - 1) // (NS * 8)) * (NS * 8)
  rpt = acc_rows // NS
  cpt = n_chunks // (NC * NS)
  nit = cpt // NBUF
  assert cpt % NBUF == 0 and NBUF == 4
  mesh = plsc.VectorSubcoreMesh(core_axis_name="c", subcore_axis_name="s")

  @functools.partial(
      pl.kernel,
      out_type=jax.ShapeDtypeStruct((NC, n_nodes, D), jnp.float32),
      mesh=mesh,
      scratch_types=[
          pltpu.VMEM((2, 2, 1, CH), jnp.int32),
          pltpu.VMEM((cpt, 1, CH), jnp.int32),
          pltpu.VMEM((2, CH, D), jnp.float32),
          pltpu.VMEM_SHARED((acc_rows, D), jnp.float32),
          [pltpu.SemaphoreType.DMA] * 2,
          [pltpu.SemaphoreType.DMA] * 2,
          [pltpu.SemaphoreType.DMA] * 2,
          pltpu.SemaphoreType.DMA,
      ],
  )
  def agg(table_hbm, zeros_hbm, src_hbm, dst_hbm, out_hbm,
          idx_sg, idx_d, rows, acc, gsem, ssem, isem, hsem):
    c = lax.axis_index("c")
    s = lax.axis_index("s")
    wid = s * NC + c
    base = wid * cpt
    ic = pltpu.async_copy(dst_hbm.at[pl.ds(base, cpt)], idx_d, hsem)
    pltpu.sync_copy(zeros_hbm, rows.at[0])
    for k in range(0, rpt, CH):
      sz = min(CH, rpt - k)
      pltpu.sync_copy(rows.at[0].at[pl.ds(0, sz)],
                      acc.at[pl.ds(s * rpt + k, sz)])
    pltpu.sync_copy(src_hbm.at[pl.ds(base, 2)], idx_sg.at[0])
    pltpu.async_copy(table_hbm.at[idx_sg.at[0, 0, 0]], rows.at[0], gsem[0])
    ic.wait()
    plsc.subcore_barrier()

    def step(i, q):
      b = q % 2
      nb = 1 - b
      j = i * NBUF + q
      if q == 0:
        pltpu.async_copy(src_hbm.at[pl.ds(base + i * NBUF + 2, 2)],
                         idx_sg.at[1], isem[1])
      if q == 2:
        @pl.when(i + 1 < nit)
        def _():
          pltpu.async_copy(src_hbm.at[pl.ds(base + (i + 1) * NBUF, 2)],
                           idx_sg.at[0], isem[0])
      pltpu.make_async_copy(zeros_hbm, rows.at[b], gsem[b]).wait()
      pltpu.async_copy(rows.at[b], acc.at[idx_d.at[j, 0]], ssem[b], add=True)
      if q == 0:
        @pl.when(i > 0)
        def _():
          pltpu.make_async_copy(zeros_hbm, rows.at[nb], ssem[nb]).wait()
        nxt = idx_sg.at[0, 1, 0]
        pltpu.async_copy(table_hbm.at[nxt], rows.at[nb], gsem[nb])
      else:
        pltpu.make_async_copy(zeros_hbm, rows.at[nb], ssem[nb]).wait()
        if q == 1:
          pltpu.make_async_copy(src_hbm.at[pl.ds(0, 2)], idx_sg.at[1],
                                isem[1]).wait()
          nxt = idx_sg.at[1, 0, 0]
          pltpu.async_copy(table_hbm.at[nxt], rows.at[nb], gsem[nb])
        elif q == 2:
          nxt = idx_sg.at[1, 1, 0]
          pltpu.async_copy(table_hbm.at[nxt], rows.at[nb], gsem[nb])
        else:
          @pl.when(i + 1 < nit)
          def _():
            pltpu.make_async_copy(src_hbm.at[pl.ds(0, 2)], idx_sg.at[0],
                                  isem[0]).wait()
            nxt = idx_sg.at[0, 0, 0]
            pltpu.async_copy(table_hbm.at[nxt], rows.at[nb], gsem[nb])

    def body(i, _):
      for q in range(NBUF):
        step(i, q)
      return 0
    lax.fori_loop(0, nit, body, 0)
    pltpu.make_async_copy(zeros_hbm, rows.at[1], ssem[1]).wait()
    plsc.subcore_barrier()
    _drain(acc, rows.at[0], out_hbm.at[c], s, n_nodes)

  return agg


def _mm_scale(degA, degB, x, w0t):
  n, d_in = x.shape
  bm = 1000
  grid = (n // bm,)

  def body(da_ref, db_ref, x_ref, w_ref, t_ref, dis_ref):
    deg = da_ref[:, :1] + db_ref[:, :1]
    dis = jnp.where(deg > 0, lax.rsqrt(deg), 0.0)
    t_ref[...] = dis * jnp.dot(x_ref[...], w_ref[...],
                               preferred_element_type=jnp.float32)
    dis_ref[...] = dis

  return pl.pallas_call(
      body,
      grid=grid,
      in_specs=[
          pl.BlockSpec((bm, D), lambda i: (i, 0)),
          pl.BlockSpec((bm, D), lambda i: (i, 0)),
          pl.BlockSpec((bm, d_in), lambda i: (i, 0)),
          pl.BlockSpec((d_in, D), lambda i: (0, 0)),
      ],
      out_specs=[
          pl.BlockSpec((bm, D), lambda i: (i, 0)),
          pl.BlockSpec((bm, 1), lambda i: (i, 0)),
      ],
      out_shape=[
          jax.ShapeDtypeStruct((n, D), jnp.float32),
          jax.ShapeDtypeStruct((n, 1), jnp.float32),
      ],
  )(degA, degB, x, w0t)


def _combine_stats(accA, accB, dis):
  n = accA.shape[0]
  bm = 1000
  grid = (n // bm,)

  def body(a_ref, b_ref, dis_ref, u_ref, st_ref):
    i = pl.program_id(0)
    u = dis_ref[...] * (a_ref[...] + b_ref[...])
    u_ref[...] = u
    @pl.when(i == 0)
    def _():
      st_ref[...] = jnp.zeros((2, D), jnp.float32)
    st_ref[0:1, :] += jnp.sum(u, axis=0, keepdims=True)
    st_ref[1:2, :] += jnp.sum(u * u, axis=0, keepdims=True)

  return pl.pallas_call(
      body,
      grid=grid,
      in_specs=[
          pl.BlockSpec((bm, D), lambda i: (i, 0)),
          pl.BlockSpec((bm, D), lambda i: (i, 0)),
          pl.BlockSpec((bm, 1), lambda i: (i, 0)),
      ],
      out_specs=[
          pl.BlockSpec((bm, D), lambda i: (i, 0)),
          pl.BlockSpec((2, D), lambda i: (0, 0)),
      ],
      out_shape=[
          jax.ShapeDtypeStruct((n, D), jnp.float32),
          jax.ShapeDtypeStruct((2, D), jnp.float32),
      ],
  )(accA, accB, dis)


def _bn_relu_mm_scale(u, st, dis, w1t):
  n = u.shape[0]
  bm = 1000
  grid = (n // bm,)

  def body(u_ref, st_ref, dis_ref, w_ref, t_ref):
    mean = st_ref[0:1, :] / n
    var = st_ref[1:2, :] / n - mean * mean
    h = jnp.maximum((u_ref[...] - mean) * lax.rsqrt(var + EPS), 0.0)
    t_ref[...] = dis_ref[...] * jnp.dot(h, w_ref[...],
                                        preferred_element_type=jnp.float32)

  return pl.pallas_call(
      body,
      grid=grid,
      in_specs=[
          pl.BlockSpec((bm, D), lambda i: (i, 0)),
          pl.BlockSpec((2, D), lambda i: (0, 0)),
          pl.BlockSpec((bm, 1), lambda i: (i, 0)),
          pl.BlockSpec((D, D), lambda i: (0, 0)),
      ],
      out_specs=pl.BlockSpec((bm, D), lambda i: (i, 0)),
      out_shape=jax.ShapeDtypeStruct((n, D), jnp.float32),
  )(u, st, dis, w1t)


def _combine_bias(accA, accB, dis, b1):
  n = accA.shape[0]
  bm = 1000
  grid = (n // bm,)

  def body(a_ref, b_ref, dis_ref, b1_ref, o_ref):
    o_ref[...] = dis_ref[...] * (a_ref[...] + b_ref[...]) + b1_ref[...]

  return pl.pallas_call(
      body,
      grid=grid,
      in_specs=[
          pl.BlockSpec((bm, D), lambda i: (i, 0)),
          pl.BlockSpec((bm, D), lambda i: (i, 0)),
          pl.BlockSpec((bm, 1), lambda i: (i, 0)),
          pl.BlockSpec((1, D), lambda i: (0, 0)),
      ],
      out_specs=pl.BlockSpec((bm, D), lambda i: (i, 0)),
      out_shape=jax.ShapeDtypeStruct((n, D), jnp.float32),
  )(accA, accB, dis, b1)


def kernel(x, edge_index, W0, b0, W1, b1):
  n, _ = x.shape
  e = edge_index.shape[1]
  src = edge_index[0]
  dst = edge_index[1]

  grp = NC * NS * NBUF
  n_chunks = ((e + CH - 1) // CH + grp - 1) // grp * grp
  e_pad = n_chunks * CH
  src_p = jnp.concatenate(
      [src, jnp.zeros((e_pad - e,), jnp.int32)]).reshape(n_chunks, 1, CH)
  dst_p = jnp.concatenate(
      [dst, jnp.full((e_pad - e,), n, jnp.int32)]).reshape(n_chunks, 1, CH)

  ones = jnp.ones((CH, D), jnp.float32)
  zeros = jnp.zeros((CH, D), jnp.float32)

  deg = _hist_kernel(n, n_chunks)(zeros, ones, dst_p)
  t0, dis = _mm_scale(deg[0], deg[1], x, W0.T)
  agg = _agg_kernel(n, n_chunks)
  acc0 = agg(t0, zeros, src_p, dst_p)
  u, st = _combine_stats(acc0[0], acc0[1], dis)
  t1 = _bn_relu_mm_scale(u, st, dis, W1.T)
  acc1 = agg(t1, zeros, src_p, dst_p)
  return _combine_bias(acc1[0], acc1[1], dis, b1.reshape(1, D))


# speedup vs baseline: 1.0246x; 1.0246x over previous
import functools

import jax
import jax.numpy as jnp
from jax import lax
from jax.experimental import pallas as pl
from jax.experimental.pallas import tpu as pltpu
from jax.experimental.pallas import tpu_sc as plsc

EPS = 1e-5
CH = 128
NC = 2
NS = 16
D = 128


def _drain(acc, stage, out_ref, s, n_nodes):
  bpt = (n_nodes // NS) // 8 * 8
  rem = n_nodes - NS * bpt
  for k in range(0, bpt, CH):
    sz = min(CH, bpt - k)
    dyn = s * bpt + k
    pltpu.sync_copy(acc.at[pl.ds(dyn, sz)], stage.at[pl.ds(0, sz)])
    pltpu.sync_copy(stage.at[pl.ds(0, sz)], out_ref.at[pl.ds(dyn, sz)])
  if rem:
    @pl.when(s == NS - 1)
    def _():
      off = NS * bpt
      pltpu.sync_copy(acc.at[pl.ds(off, rem)], stage.at[pl.ds(0, rem)])
      pltpu.sync_copy(stage.at[pl.ds(0, rem)], out_ref.at[pl.ds(off, rem)])


def _hist_kernel(n_nodes, n_chunks):
  acc_rows = ((n_nodes + 16 + NS * 8 - 1) // (NS * 8)) * (NS * 8)
  rpt = acc_rows // NS
  cpt = n_chunks // (NC * NS)
  mesh = plsc.VectorSubcoreMesh(core_axis_name="c", subcore_axis_name="s")

  depth = 4
  @functools.partial(
      pl.kernel,
      out_type=jax.ShapeDtypeStruct((NC, n_nodes, D), jnp.float32),
      mesh=mesh,
      scratch_types=[
          pltpu.VMEM((cpt, 1, CH), jnp.int32),
          pltpu.VMEM((CH, D), jnp.float32),
          pltpu.VMEM_SHARED((acc_rows, D), jnp.float32),
          pltpu.SemaphoreType.DMA,
          pltpu.SemaphoreType.DMA,
      ],
  )
  def hist(zeros_hbm, ones_hbm, dst_hbm, out_hbm, idx_d, rows, acc,
           ssem, hsem):
    c = lax.axis_index("c")
    s = lax.axis_index("s")
    wid = s * NC + c
    ic = pltpu.async_copy(dst_hbm.at[pl.ds(wid * cpt, cpt)], idx_d, hsem)
    pltpu.sync_copy(zeros_hbm, rows)
    for k in range(0, rpt, CH):
      sz = min(CH, rpt - k)
      pltpu.sync_copy(rows.at[pl.ds(0, sz)],
                      acc.at[pl.ds(s * rpt + k, sz)])
    pltpu.sync_copy(ones_hbm, rows)
    ic.wait()
    plsc.subcore_barrier()

    def body(j, _):
      pltpu.async_copy(rows, acc.at[idx_d.at[j, 0]], ssem, add=True)
      @pl.when(j >= depth)
      def _():
        pltpu.make_async_copy(zeros_hbm, rows, ssem).wait()
      return 0
    lax.fori_loop(0, cpt, body, 0)
    for _ in range(depth):
      pltpu.make_async_copy(zeros_hbm, rows, ssem).wait()
    plsc.subcore_barrier()
    _drain(acc, rows, out_hbm.at[c], s, n_nodes)

  return hist


NBUF = 4


def _agg_kernel(n_nodes, n_chunks):
  acc_rows = ((n_nodes + 16 + NS * 8 - 1) // (NS * 8)) * (NS * 8)
  rpt = acc_rows // NS
  cpt = n_chunks // (NC * NS)
  nit = cpt // NBUF
  assert cpt % NBUF == 0 and NBUF == 4
  mesh = plsc.VectorSubcoreMesh(core_axis_name="c", subcore_axis_name="s")

  @functools.partial(
      pl.kernel,
      out_type=jax.ShapeDtypeStruct((NC, n_nodes, D), jnp.float32),
      mesh=mesh,
      scratch_types=[
          pltpu.VMEM((2, 2, 1, CH), jnp.int32),
          pltpu.VMEM((cpt, 1, CH), jnp.int32),
          pltpu.VMEM((2, CH, D), jnp.float32),
          pltpu.VMEM_SHARED((acc_rows, D), jnp.float32),
          [pltpu.SemaphoreType.DMA] * 2,
          [pltpu.SemaphoreType.DMA] * 2,
          pltpu.SemaphoreType.DMA,
      ],
  )
  def agg(table_hbm, zeros_hbm, src_hbm, dst_hbm, out_hbm,
          idx_sg, idx_d, rows, acc, gsem, isem, hsem):
    c = lax.axis_index("c")
    s = lax.axis_index("s")
    wid = s * NC + c
    base = wid * cpt
    ic = pltpu.async_copy(dst_hbm.at[pl.ds(base, cpt)], idx_d, hsem)
    pltpu.sync_copy(zeros_hbm, rows.at[0])
    for k in range(0, rpt, CH):
      sz = min(CH, rpt - k)
      pltpu.sync_copy(rows.at[0].at[pl.ds(0, sz)],
                      acc.at[pl.ds(s * rpt + k, sz)])
    pltpu.sync_copy(src_hbm.at[pl.ds(base, 2)], idx_sg.at[0])
    pltpu.async_copy(table_hbm.at[idx_sg.at[0, 0, 0]], rows.at[0], gsem[0])
    ic.wait()
    plsc.subcore_barrier()

    def step(i, q):
      b = q % 2
      nb = 1 - b
      j = i * NBUF + q
      if q == 0:
        pltpu.async_copy(src_hbm.at[pl.ds(base + i * NBUF + 2, 2)],
                         idx_sg.at[1], isem[1])
      if q == 2:
        @pl.when(i + 1 < nit)
        def _():
          pltpu.async_copy(src_hbm.at[pl.ds(base + (i + 1) * NBUF, 2)],
                           idx_sg.at[0], isem[0])
      pltpu.make_async_copy(zeros_hbm, rows.at[b], gsem[b]).wait()
      if q == 0:
        nxt = idx_sg.at[0, 1, 0]
        pltpu.async_copy(table_hbm.at[nxt], rows.at[nb], gsem[nb])
      elif q == 1:
        pltpu.make_async_copy(src_hbm.at[pl.ds(0, 2)], idx_sg.at[1],
                              isem[1]).wait()
        nxt = idx_sg.at[1, 0, 0]
        pltpu.async_copy(table_hbm.at[nxt], rows.at[nb], gsem[nb])
      elif q == 2:
        nxt = idx_sg.at[1, 1, 0]
        pltpu.async_copy(table_hbm.at[nxt], rows.at[nb], gsem[nb])
      else:
        @pl.when(i + 1 < nit)
        def _():
          pltpu.make_async_copy(src_hbm.at[pl.ds(0, 2)], idx_sg.at[0],
                                isem[0]).wait()
          nxt = idx_sg.at[0, 0, 0]
          pltpu.async_copy(table_hbm.at[nxt], rows.at[nb], gsem[nb])
      pltpu.sync_copy(rows.at[b], acc.at[idx_d.at[j, 0]], add=True)

    def body(i, _):
      for q in range(NBUF):
        step(i, q)
      return 0
    lax.fori_loop(0, nit, body, 0)
    plsc.subcore_barrier()
    _drain(acc, rows.at[0], out_hbm.at[c], s, n_nodes)

  return agg


def _mm_scale(degA, degB, x, w0t):
  n, d_in = x.shape
  bm = 1000
  grid = (n // bm,)

  def body(da_ref, db_ref, x_ref, w_ref, t_ref, dis_ref):
    deg = da_ref[:, :1] + db_ref[:, :1]
    dis = jnp.where(deg > 0, lax.rsqrt(deg), 0.0)
    t_ref[...] = dis * jnp.dot(x_ref[...], w_ref[...],
                               preferred_element_type=jnp.float32)
    dis_ref[...] = dis

  return pl.pallas_call(
      body,
      grid=grid,
      in_specs=[
          pl.BlockSpec((bm, D), lambda i: (i, 0)),
          pl.BlockSpec((bm, D), lambda i: (i, 0)),
          pl.BlockSpec((bm, d_in), lambda i: (i, 0)),
          pl.BlockSpec((d_in, D), lambda i: (0, 0)),
      ],
      out_specs=[
          pl.BlockSpec((bm, D), lambda i: (i, 0)),
          pl.BlockSpec((bm, 1), lambda i: (i, 0)),
      ],
      out_shape=[
          jax.ShapeDtypeStruct((n, D), jnp.float32),
          jax.ShapeDtypeStruct((n, 1), jnp.float32),
      ],
  )(degA, degB, x, w0t)


def _combine_stats(accA, accB, dis):
  n = accA.shape[0]
  bm = 1000
  grid = (n // bm,)

  def body(a_ref, b_ref, dis_ref, u_ref, st_ref):
    i = pl.program_id(0)
    u = dis_ref[...] * (a_ref[...] + b_ref[...])
    u_ref[...] = u
    @pl.when(i == 0)
    def _():
      st_ref[...] = jnp.zeros((2, D), jnp.float32)
    st_ref[0:1, :] += jnp.sum(u, axis=0, keepdims=True)
    st_ref[1:2, :] += jnp.sum(u * u, axis=0, keepdims=True)

  return pl.pallas_call(
      body,
      grid=grid,
      in_specs=[
          pl.BlockSpec((bm, D), lambda i: (i, 0)),
          pl.BlockSpec((bm, D), lambda i: (i, 0)),
          pl.BlockSpec((bm, 1), lambda i: (i, 0)),
      ],
      out_specs=[
          pl.BlockSpec((bm, D), lambda i: (i, 0)),
          pl.BlockSpec((2, D), lambda i: (0, 0)),
      ],
      out_shape=[
          jax.ShapeDtypeStruct((n, D), jnp.float32),
          jax.ShapeDtypeStruct((2, D), jnp.float32),
      ],
  )(accA, accB, dis)


def _bn_relu_mm_scale(u, st, dis, w1t):
  n = u.shape[0]
  bm = 1000
  grid = (n // bm,)

  def body(u_ref, st_ref, dis_ref, w_ref, t_ref):
    mean = st_ref[0:1, :] / n
    var = st_ref[1:2, :] / n - mean * mean
    h = jnp.maximum((u_ref[...] - mean) * lax.rsqrt(var + EPS), 0.0)
    t_ref[...] = dis_ref[...] * jnp.dot(h, w_ref[...],
                                        preferred_element_type=jnp.float32)

  return pl.pallas_call(
      body,
      grid=grid,
      in_specs=[
          pl.BlockSpec((bm, D), lambda i: (i, 0)),
          pl.BlockSpec((2, D), lambda i: (0, 0)),
          pl.BlockSpec((bm, 1), lambda i: (i, 0)),
          pl.BlockSpec((D, D), lambda i: (0, 0)),
      ],
      out_specs=pl.BlockSpec((bm, D), lambda i: (i, 0)),
      out_shape=jax.ShapeDtypeStruct((n, D), jnp.float32),
  )(u, st, dis, w1t)


def _combine_bias(accA, accB, dis, b1):
  n = accA.shape[0]
  bm = 1000
  grid = (n // bm,)

  def body(a_ref, b_ref, dis_ref, b1_ref, o_ref):
    o_ref[...] = dis_ref[...] * (a_ref[...] + b_ref[...]) + b1_ref[...]

  return pl.pallas_call(
      body,
      grid=grid,
      in_specs=[
          pl.BlockSpec((bm, D), lambda i: (i, 0)),
          pl.BlockSpec((bm, D), lambda i: (i, 0)),
          pl.BlockSpec((bm, 1), lambda i: (i, 0)),
          pl.BlockSpec((1, D), lambda i: (0, 0)),
      ],
      out_specs=pl.BlockSpec((bm, D), lambda i: (i, 0)),
      out_shape=jax.ShapeDtypeStruct((n, D), jnp.float32),
  )(accA, accB, dis, b1)


def kernel(x, edge_index, W0, b0, W1, b1):
  n, _ = x.shape
  e = edge_index.shape[1]
  src = edge_index[0]
  dst = edge_index[1]

  grp = NC * NS * NBUF
  n_chunks = ((e + CH - 1) // CH + grp - 1) // grp * grp
  e_pad = n_chunks * CH
  src_p = jnp.concatenate(
      [src, jnp.zeros((e_pad - e,), jnp.int32)]).reshape(n_chunks, 1, CH)
  dst_p = jnp.concatenate(
      [dst, jnp.full((e_pad - e,), n, jnp.int32)]).reshape(n_chunks, 1, CH)

  ones = jnp.ones((CH, D), jnp.float32)
  zeros = jnp.zeros((CH, D), jnp.float32)

  deg = _hist_kernel(n, n_chunks)(zeros, ones, dst_p)
  t0, dis = _mm_scale(deg[0], deg[1], x, W0.T)
  agg = _agg_kernel(n, n_chunks)
  acc0 = agg(t0, zeros, src_p, dst_p)
  u, st = _combine_stats(acc0[0], acc0[1], dis)
  t1 = _bn_relu_mm_scale(u, st, dis, W1.T)
  acc1 = agg(t1, zeros, src_p, dst_p)
  return _combine_bias(acc1[0], acc1[1], dis, b1.reshape(1, D))
